# Initial kernel scaffold; baseline (speedup 1.0000x reference)
import jax
import jax.numpy as jnp
from jax.experimental import pallas as pl


def kernel(x_pfc, x_clus, x_glob, batch_pfc, batch_clus, batch_glob, W_p1, b_p1, W_p2, b_p2, W_c1, b_c1, W_c2, b_c2, W_e, b_e, W_o1, b_o1, W_o2, b_o2, W_o3, b_o3, W_o4, b_o4):
    raise NotImplementedError("write your pallas kernel here")


# speedup vs baseline: 11.5063x; 11.5063x over previous
import functools

import jax
import jax.numpy as jnp
from jax import lax
from jax.experimental import pallas as pl
from jax.experimental.pallas import tpu as pltpu
from jax.experimental.pallas import tpu_sc as plsc

HID = 128
K = 8
TT = 256
SCW = 512
AW = HID + 1 + 16
BIG = 1e30
IBIG = 2 ** 30
NGRAPH = 16
_SC_CORES = 2
_SC_SUBCORES = 16
_NW = _SC_CORES * _SC_SUBCORES


def _elu(x):
    return jnp.where(x > 0, x, jnp.exp(jnp.where(x > 0, 0.0, x)) - 1.0)


def _enc_p_body(x_ref, bt_ref, wp1_ref, bp1_ref, wp2_ref, bp2_ref, we_ref,
                be_ref, aug_ref, a_ref):
    x = x_ref[...]
    h = _elu(jnp.dot(x, wp1_ref[...], preferred_element_type=jnp.float32)
             + bp1_ref[...])
    xp = _elu(jnp.dot(h, wp2_ref[...], preferred_element_type=jnp.float32)
              + bp2_ref[...])
    wd = we_ref[:HID, :] - we_ref[HID:, :]
    a_ref[...] = jnp.dot(xp, wd, preferred_element_type=jnp.float32) + be_ref[...]
    bt = bt_ref[...]
    onehot = (bt == lax.broadcasted_iota(jnp.int32, (TT, NGRAPH), 1)
              ).astype(jnp.float32)
    ones = jnp.ones((TT, 1), jnp.float32)
    aug_ref[...] = jnp.concatenate([xp, ones, BIG * onehot], axis=1)


def _enc_c_body(x_ref, bc_ref, wc1_ref, bc1_ref, wc2_ref, bc2_ref, we_ref,
                aug_ref, s_ref):
    x = x_ref[...]
    h = _elu(jnp.dot(x, wc1_ref[...], preferred_element_type=jnp.float32)
             + bc1_ref[...])
    xc = _elu(jnp.dot(h, wc2_ref[...], preferred_element_type=jnp.float32)
              + bc2_ref[...])
    s_ref[...] = jnp.dot(xc, we_ref[HID:, :], preferred_element_type=jnp.float32)
    s2 = jnp.sum(xc * xc, axis=1, keepdims=True)
    bc = bc_ref[...]
    onehot = (bc == lax.broadcasted_iota(jnp.int32, (TT, NGRAPH), 1)
              ).astype(jnp.float32)
    aug_ref[...] = jnp.concatenate([xc, -0.5 * s2, -0.5 * (1.0 - onehot)],
                                   axis=1)


def _const_spec(shape):
    return pl.BlockSpec(shape, lambda i: (0,) * len(shape))


def _enc_p(x_pfc, bt_col, W_p1, b_p1, W_p2, b_p2, W_e, b_e, interpret=False):
    n = x_pfc.shape[0]
    grid = (n // TT,)
    return pl.pallas_call(
        _enc_p_body,
        grid=grid,
        in_specs=[
            pl.BlockSpec((TT, 8), lambda i: (i, 0)),
            pl.BlockSpec((TT, 1), lambda i: (i, 0)),
            _const_spec((8, HID)),
            _const_spec((1, HID)),
            _const_spec((HID, HID)),
            _const_spec((1, HID)),
            _const_spec((2 * HID, HID)),
            _const_spec((1, HID)),
        ],
        out_specs=[
            pl.BlockSpec((TT, AW), lambda i: (i, 0)),
            pl.BlockSpec((TT, HID), lambda i: (i, 0)),
        ],
        out_shape=[
            jax.ShapeDtypeStruct((n, AW), jnp.float32),
            jax.ShapeDtypeStruct((n, HID), jnp.float32),
        ],
        interpret=interpret,
    )(x_pfc, bt_col, W_p1, b_p1, W_p2, b_p2, W_e, b_e)


def _enc_c(x_clus, bc_col, W_c1, b_c1, W_c2, b_c2, W_e, interpret=False):
    n = x_clus.shape[0]
    grid = (n // TT,)
    return pl.pallas_call(
        _enc_c_body,
        grid=grid,
        in_specs=[
            pl.BlockSpec((TT, 4), lambda i: (i, 0)),
            pl.BlockSpec((TT, 1), lambda i: (i, 0)),
            _const_spec((4, HID)),
            _const_spec((1, HID)),
            _const_spec((HID, HID)),
            _const_spec((1, HID)),
            _const_spec((2 * HID, HID)),
        ],
        out_specs=[
            pl.BlockSpec((TT, AW), lambda i: (i, 0)),
            pl.BlockSpec((TT, HID), lambda i: (i, 0)),
        ],
        out_shape=[
            jax.ShapeDtypeStruct((n, AW), jnp.float32),
            jax.ShapeDtypeStruct((n, HID), jnp.float32),
        ],
        interpret=interpret,
    )(x_clus, bc_col, W_c1, b_c1, W_c2, b_c2, W_e)


def _make_knn_body(n_src):
    n_ch = n_src // SCW

    def body(tmin_ref, tmax_ref, cmin_ref, cmax_ref, tgt_ref, src_ref,
             idx_ref):
        i = pl.program_id(0)
        t = tgt_ref[...]
        tf = t[:, :HID]
        t2 = jnp.sum(tf * tf, axis=1, keepdims=True)
        bt_lo = tmin_ref[i]
        bt_hi = tmax_ref[i]

        bd0 = jnp.full((TT, K), jnp.inf, jnp.float32)
        bi0 = jnp.full((TT, K), IBIG, jnp.int32)

        def step(c, st):
            def do(st):
                bd, bi = st
                s = src_ref[pl.ds(c * SCW, SCW), :]
                m = lax.dot_general(t, s, (((1,), (1,)), ((), ())),
                                    preferred_element_type=jnp.float32)
                d = t2 - 2.0 * m
                thr = bd[:, K - 1:K]
                imp = jnp.any(jnp.min(d, axis=1, keepdims=True) < thr)

                def merge(args):
                    bd, bi, d = args
                    ii = (lax.broadcasted_iota(jnp.int32, (TT, SCW), 1)
                          + c * SCW)
                    cd = jnp.concatenate([bd, d], axis=1)
                    ci = jnp.concatenate([bi, ii], axis=1)
                    nd, ni = [], []
                    for _ in range(K):
                        mn = jnp.min(cd, axis=1, keepdims=True)
                        sel = jnp.min(jnp.where(cd == mn, ci, IBIG), axis=1,
                                      keepdims=True)
                        nd.append(mn)
                        ni.append(sel)
                        cd = jnp.where((cd == mn) & (ci == sel), jnp.inf, cd)
                    return (jnp.concatenate(nd, axis=1),
                            jnp.concatenate(ni, axis=1))

                return lax.cond(imp, merge, lambda a: (a[0], a[1]),
                                (bd, bi, d))

            ov = jnp.logical_and(cmin_ref[c] <= bt_hi, cmax_ref[c] >= bt_lo)
            return lax.cond(ov, do, lambda x: x, st)

        bd, bi = lax.fori_loop(0, n_ch, step, (bd0, bi0))
        idx_ref[...] = jnp.minimum(bi, n_src - 1)

    return body


def _knn(tgt_aug, src_aug, tmin, tmax, cmin, cmax, interpret=False):
    n_tgt = tgt_aug.shape[0]
    n_src = src_aug.shape[0]
    grid = (n_tgt // TT,)
    return pl.pallas_call(
        _make_knn_body(n_src),
        grid_spec=pltpu.PrefetchScalarGridSpec(
            num_scalar_prefetch=4,
            grid=grid,
            in_specs=[
                pl.BlockSpec((TT, AW), lambda i, *_: (i, 0)),
                pl.BlockSpec((n_src, AW), lambda i, *_: (0, 0)),
            ],
            out_specs=pl.BlockSpec((TT, K), lambda i, *_: (i, 0)),
        ),
        out_shape=jax.ShapeDtypeStruct((n_tgt, K), jnp.int32),
        interpret=interpret,
    )(tmin, tmax, cmin, cmax, tgt_aug, src_aug)


def _sc_gather(table, idx_flat):
    n_rows = idx_flat.shape[0]
    d = table.shape[1]
    per_w = n_rows // _NW
    ch = 128
    n_ch = per_w // ch
    mesh = plsc.VectorSubcoreMesh(core_axis_name="c", subcore_axis_name="s")

    @functools.partial(
        pl.kernel,
        mesh=mesh,
        out_type=jax.ShapeDtypeStruct((n_rows, d), jnp.float32),
        scratch_types=[
            pltpu.VMEM((ch,), jnp.int32),
            pltpu.VMEM((ch, d), jnp.float32),
            pltpu.SemaphoreType.DMA,
        ],
    )
    def k(table_hbm, idx_hbm, out_hbm, idx_v, rows_v, sem):
        wid = lax.axis_index("s") * _SC_CORES + lax.axis_index("c")
        base = wid * per_w

        def step(j, carry):
            off = base + j * ch
            pltpu.sync_copy(idx_hbm.at[pl.ds(off, ch)], idx_v)
            pltpu.async_copy(table_hbm.at[idx_v], rows_v, sem).wait()
            pltpu.sync_copy(rows_v, out_hbm.at[pl.ds(off, ch)])
            return carry

        lax.fori_loop(0, n_ch, step, 0)

    return k(table, idx_flat)


def _combine1_body(bt_ref, a_ref, g_ref, we_ref, aug_ref, s2_ref):
    a = a_ref[...]
    f = _elu(a + g_ref[0])
    for kk in range(1, K):
        f = jnp.maximum(f, _elu(a + g_ref[kk]))
    s2_ref[...] = jnp.dot(f, we_ref[HID:, :],
                          preferred_element_type=jnp.float32)
    sq = jnp.sum(f * f, axis=1, keepdims=True)
    bt = bt_ref[...]
    onehot = (bt == lax.broadcasted_iota(jnp.int32, (TT, NGRAPH), 1)
              ).astype(jnp.float32)
    aug_ref[...] = jnp.concatenate([f, -0.5 * sq, -0.5 * (1.0 - onehot)],
                                   axis=1)


def _combine1(bt_col, A, G, W_e, interpret=False):
    n = A.shape[0]
    grid = (n // TT,)
    return pl.pallas_call(
        _combine1_body,
        grid=grid,
        in_specs=[
            pl.BlockSpec((TT, 1), lambda i: (i, 0)),
            pl.BlockSpec((TT, HID), lambda i: (i, 0)),
            pl.BlockSpec((K, TT, HID), lambda i: (0, i, 0)),
            _const_spec((2 * HID, HID)),
        ],
        out_specs=[
            pl.BlockSpec((TT, AW), lambda i: (i, 0)),
            pl.BlockSpec((TT, HID), lambda i: (i, 0)),
        ],
        out_shape=[
            jax.ShapeDtypeStruct((n, AW), jnp.float32),
            jax.ShapeDtypeStruct((n, HID), jnp.float32),
        ],
        interpret=interpret,
    )(bt_col, A, G, W_e)


def _final_body(a_ref, g_ref, w1_ref, b1_ref, w2_ref, b2_ref, w3_ref, b3_ref,
                w4_ref, b4_ref, out_ref):
    a = a_ref[...]
    f = _elu(a + g_ref[0])
    for kk in range(1, K):
        f = jnp.maximum(f, _elu(a + g_ref[kk]))
    h = _elu(jnp.dot(f, w1_ref[...], preferred_element_type=jnp.float32)
             + b1_ref[...])
    h = _elu(jnp.dot(h, w2_ref[...], preferred_element_type=jnp.float32)
             + b2_ref[...])
    h = _elu(jnp.dot(h, w3_ref[...], preferred_element_type=jnp.float32)
             + b3_ref[...])
    z = jnp.dot(h, w4_ref[...], preferred_element_type=jnp.float32) + b4_ref[...]
    out_ref[...] = jax.nn.sigmoid(z)


def _final(A, G, W_o1, b_o1, W_o2, b_o2, W_o3, b_o3, W_o4, b_o4,
           interpret=False):
    n = A.shape[0]
    grid = (n // TT,)
    return pl.pallas_call(
        _final_body,
        grid=grid,
        in_specs=[
            pl.BlockSpec((TT, HID), lambda i: (i, 0)),
            pl.BlockSpec((K, TT, HID), lambda i: (0, i, 0)),
            _const_spec((HID, 64)),
            _const_spec((1, 64)),
            _const_spec((64, 32)),
            _const_spec((1, 32)),
            _const_spec((32, 4)),
            _const_spec((1, 4)),
            _const_spec((4, 1)),
            _const_spec((1, 1)),
        ],
        out_specs=pl.BlockSpec((TT, 1), lambda i: (i, 0)),
        out_shape=jax.ShapeDtypeStruct((n, 1), jnp.float32),
        interpret=interpret,
    )(A, G, W_o1, b_o1, W_o2, b_o2, W_o3, b_o3, W_o4, b_o4)


def _chunk_minmax(b, width):
    r = b.reshape(-1, width)
    return jnp.min(r, axis=1), jnp.max(r, axis=1)


def _pipeline(x_pfc, x_clus, batch_pfc, batch_clus, W_p1, b_p1, W_p2, b_p2,
              W_c1, b_c1, W_c2, b_c2, W_e, b_e, W_o1, b_o1, W_o2, b_o2,
              W_o3, b_o3, W_o4, b_o4, gather_fn, interpret=False):
    n_p = x_pfc.shape[0]
    n_c = x_clus.shape[0]
    row = lambda v: v.reshape(1, -1)
    bt_col = batch_pfc.reshape(n_p, 1)
    bc_col = batch_clus.reshape(n_c, 1)

    tmin, tmax = _chunk_minmax(batch_pfc, TT)
    cmin1, cmax1 = _chunk_minmax(batch_clus, SCW)
    cmin2, cmax2 = _chunk_minmax(batch_pfc, SCW)

    tgt_aug, A = _enc_p(x_pfc, bt_col, W_p1, row(b_p1), W_p2, row(b_p2),
                        W_e, row(b_e), interpret=interpret)
    src_aug1, S1 = _enc_c(x_clus, bc_col, W_c1, row(b_c1), W_c2, row(b_c2),
                          W_e, interpret=interpret)

    idx1 = _knn(tgt_aug, src_aug1, tmin, tmax, cmin1, cmax1,
                interpret=interpret)
    g1 = gather_fn(S1, idx1.T.reshape(-1))
    f1_aug, S2 = _combine1(bt_col, A, g1.reshape(K, n_p, HID), W_e,
                           interpret=interpret)

    idx2 = _knn(tgt_aug, f1_aug, tmin, tmax, cmin2, cmax2,
                interpret=interpret)
    g2 = gather_fn(S2, idx2.T.reshape(-1))
    out = _final(A, g2.reshape(K, n_p, HID), W_o1, row(b_o1), W_o2, row(b_o2),
                 W_o3, row(b_o3), W_o4, row(b_o4), interpret=interpret)
    return out


def kernel(x_pfc, x_clus, x_glob, batch_pfc, batch_clus, batch_glob,
           W_p1, b_p1, W_p2, b_p2, W_c1, b_c1, W_c2, b_c2, W_e, b_e,
           W_o1, b_o1, W_o2, b_o2, W_o3, b_o3, W_o4, b_o4):
    out = _pipeline(x_pfc, x_clus, batch_pfc, batch_clus, W_p1, b_p1, W_p2,
                    b_p2, W_c1, b_c1, W_c2, b_c2, W_e, b_e, W_o1, b_o1,
                    W_o2, b_o2, W_o3, b_o3, W_o4, b_o4, _sc_gather)
    return (out, batch_pfc)
